# Initial kernel scaffold; baseline (speedup 1.0000x reference)
import jax
import jax.numpy as jnp
from jax.experimental import pallas as pl


def kernel(x, knots, poly_params, mixture_weights, integrals_2dgrid):
    raise NotImplementedError("write your pallas kernel here")


# speedup vs baseline: 90.0787x; 90.0787x over previous
import functools

import jax
import jax.numpy as jnp
from jax import lax
from jax.experimental import pallas as pl
from jax.experimental.pallas import tpu as pltpu
from jax.experimental.pallas import tpu_sc as plsc

_C = 2048
_L = 16
_ROW = 48
_G = _C // _L
_STREAM = 128

_LN2 = 0.6931471805599453


def _density_log(acc):
    d = jnp.maximum(acc, jnp.float32(1e-38)) * jnp.float32(16777216.0)
    b = lax.bitcast_convert_type(d, jnp.int32)
    e = lax.shift_right_arithmetic(b, 23) - (127 + 24)
    mb = jnp.bitwise_or(jnp.bitwise_and(b, 0x007FFFFF), 0x3F800000)
    m = lax.bitcast_convert_type(mb, jnp.float32)
    big = m > jnp.float32(1.5)
    m = jnp.where(big, m * jnp.float32(0.5), m)
    e = jnp.where(big, e + 1, e)
    r = (m - jnp.float32(1.0)) / (m + jnp.float32(1.0))
    r2 = r * r
    poly = jnp.float32(1.0) + r2 * (
        jnp.float32(1.0 / 3.0)
        + r2 * (jnp.float32(0.2) + r2 * jnp.float32(1.0 / 7.0)))
    return e.astype(jnp.float32) * jnp.float32(_LN2) + jnp.float32(2.0) * r * poly


def _make_sc_kernel(n, kn, bb):
    num_chunks = (n + _C - 1) // _C
    info = plsc.get_sparse_core_info()
    nw = info.num_cores * info.num_subcores

    @functools.partial(
        pl.kernel,
        out_type=jax.ShapeDtypeStruct((n,), jnp.float32),
        mesh=plsc.VectorSubcoreMesh(core_axis_name="c", subcore_axis_name="s"),
        compiler_params=pltpu.CompilerParams(
            use_tc_tiling_on_sc=False, needs_layout_passes=False),
        scratch_types=[
            pltpu.VMEM((kn, 2), jnp.float32),
            pltpu.VMEM((_C, 2), jnp.float32),
            pltpu.VMEM((_C,), jnp.float32),
            pltpu.VMEM((_C,), jnp.float32),
            pltpu.VMEM((_C,), jnp.int32),
            pltpu.VMEM((_C, _ROW), jnp.float32),
            pltpu.VMEM((_C,), jnp.float32),
            pltpu.SemaphoreType.DMA,
        ],
    )
    def sc_kernel(x_hbm, knots_hbm, packed_hbm, out_hbm,
                  knots_v, x_v, t0_v, t1_v, idx_v, rows_v, outc_v, sem):
        wid = lax.axis_index("s") * info.num_cores + lax.axis_index("c")
        pltpu.sync_copy(knots_hbm, knots_v)

        iota = lax.broadcasted_iota(jnp.int32, (_L,), 0)
        zero16 = jnp.zeros((_L,), jnp.int32)
        one16 = jnp.ones((_L,), jnp.int32)
        b_max = bb - 1

        def do_chunk(k, _):
            c = wid + k * nw
            base = jnp.minimum(c * _C, n - _C)
            pltpu.sync_copy(x_hbm.at[pl.ds(base, _C), :], x_v)

            def grp_a(g, _):
                p = g * _L + iota
                x0 = plsc.load_gather(x_v, [p, zero16])
                x1 = plsc.load_gather(x_v, [p, one16])
                pos0 = jnp.zeros((_L,), jnp.int32)
                pos1 = jnp.zeros((_L,), jnp.int32)
                for s in (256, 128, 64, 32, 16, 8, 4, 2, 1):
                    k0 = plsc.load_gather(knots_v, [pos0 + (s - 1), zero16])
                    pos0 = jnp.where(k0 < x0, pos0 + s, pos0)
                    k1 = plsc.load_gather(knots_v, [pos1 + (s - 1), one16])
                    pos1 = jnp.where(k1 < x1, pos1 + s, pos1)
                i0 = jnp.clip(pos0 - 1, 0, b_max)
                i1 = jnp.clip(pos1 - 1, 0, b_max)
                t0 = x0 - plsc.load_gather(knots_v, [i0, zero16])
                t1 = x1 - plsc.load_gather(knots_v, [i1, one16])
                t0_v[pl.ds(g * _L, _L)] = t0
                t1_v[pl.ds(g * _L, _L)] = t1
                idx_v[pl.ds(g * _L, _L)] = i0 * bb + i1
                return 0

            lax.fori_loop(0, _G, grp_a, 0)

            descs = []
            for s in range(_C // _STREAM):
                descs.append(pltpu.async_copy(
                    packed_hbm.at[idx_v.at[pl.ds(s * _STREAM, _STREAM)]],
                    rows_v.at[pl.ds(s * _STREAM, _STREAM)],
                    sem))
            for d in descs:
                d.wait()

            def grp_c(g, _):
                p = g * _L + iota
                t0 = t0_v[pl.ds(g * _L, _L)]
                t1 = t1_v[pl.ds(g * _L, _L)]
                acc = jnp.zeros((_L,), jnp.float32)
                for m in range(4):
                    a = []
                    for i in range(3):
                        col = m * 9 + i * 3
                        c0 = plsc.load_gather(rows_v, [p, jnp.full((_L,), col, jnp.int32)])
                        c1 = plsc.load_gather(rows_v, [p, jnp.full((_L,), col + 1, jnp.int32)])
                        c2 = plsc.load_gather(rows_v, [p, jnp.full((_L,), col + 2, jnp.int32)])
                        a.append((c2 * t1 + c1) * t1 + c0)
                    val = (a[2] * t0 + a[1]) * t0 + a[0]
                    wi = plsc.load_gather(rows_v, [p, jnp.full((_L,), 36 + m, jnp.int32)])
                    acc = acc + val * val * wi
                outc_v[pl.ds(g * _L, _L)] = _density_log(acc)
                return 0

            lax.fori_loop(0, _G, grp_c, 0)

            pltpu.sync_copy(outc_v, out_hbm.at[pl.ds(base, _C)])
            return 0

        my_chunks = (num_chunks - wid + nw - 1) // nw
        lax.fori_loop(0, my_chunks, do_chunk, 0)

    return sc_kernel


def kernel(x, knots, poly_params, mixture_weights, integrals_2dgrid):
    n = x.shape[0]
    kn = knots.shape[0]
    bb = kn - 1
    coeff = jnp.transpose(poly_params[0].reshape(4, bb * bb, 9), (1, 0, 2))
    coeff = coeff.reshape(bb * bb, 36)
    wi = mixture_weights[0][:, None] / integrals_2dgrid[0].reshape(4, bb * bb)
    wi = jnp.transpose(wi, (1, 0))
    packed = jnp.concatenate(
        [coeff, wi, jnp.zeros((bb * bb, _ROW - 40), jnp.float32)], axis=1)
    return _make_sc_kernel(n, kn, bb)(x, knots, packed)


# speedup vs baseline: 93.8799x; 1.0422x over previous
import functools

import jax
import jax.numpy as jnp
from jax import lax
from jax.experimental import pallas as pl
from jax.experimental.pallas import tpu as pltpu
from jax.experimental.pallas import tpu_sc as plsc

_C = 2048
_L = 16
_ROW = 48
_G = _C // _L
_STREAM = 128

_LN2 = 0.6931471805599453


def _density_log(acc):
    d = jnp.maximum(acc, jnp.float32(1e-38)) * jnp.float32(16777216.0)
    b = lax.bitcast_convert_type(d, jnp.int32)
    e = lax.shift_right_arithmetic(b, 23) - (127 + 24)
    mb = jnp.bitwise_or(jnp.bitwise_and(b, 0x007FFFFF), 0x3F800000)
    m = lax.bitcast_convert_type(mb, jnp.float32)
    big = m > jnp.float32(1.5)
    m = jnp.where(big, m * jnp.float32(0.5), m)
    e = jnp.where(big, e + 1, e)
    r = (m - jnp.float32(1.0)) / (m + jnp.float32(1.0))
    r2 = r * r
    poly = jnp.float32(1.0) + r2 * (
        jnp.float32(1.0 / 3.0)
        + r2 * (jnp.float32(0.2) + r2 * jnp.float32(1.0 / 7.0)))
    return e.astype(jnp.float32) * jnp.float32(_LN2) + jnp.float32(2.0) * r * poly


def _make_sc_kernel(n, kn, bb):
    num_chunks = (n + _C - 1) // _C
    info = plsc.get_sparse_core_info()
    nw = info.num_cores * info.num_subcores

    @functools.partial(
        pl.kernel,
        out_type=jax.ShapeDtypeStruct((n,), jnp.float32),
        mesh=plsc.VectorSubcoreMesh(core_axis_name="c", subcore_axis_name="s"),
        compiler_params=pltpu.CompilerParams(
            use_tc_tiling_on_sc=False, needs_layout_passes=False),
        scratch_types=[
            pltpu.VMEM((kn, 2), jnp.float32),
            pltpu.VMEM((_C, 2), jnp.float32),
            pltpu.VMEM((_C,), jnp.float32),
            pltpu.VMEM((_C,), jnp.float32),
            pltpu.VMEM((_C,), jnp.int32),
            pltpu.VMEM((_C, _ROW), jnp.float32),
            pltpu.VMEM((_C,), jnp.float32),
            pltpu.SemaphoreType.DMA,
        ],
    )
    def sc_kernel(x_hbm, knots_hbm, packed_hbm, out_hbm,
                  knots_v, x_v, t0_v, t1_v, idx_v, rows_v, outc_v, sem):
        wid = lax.axis_index("s") * info.num_cores + lax.axis_index("c")
        pltpu.sync_copy(knots_hbm, knots_v)

        iota = lax.broadcasted_iota(jnp.int32, (_L,), 0)
        zero16 = jnp.zeros((_L,), jnp.int32)
        one16 = jnp.ones((_L,), jnp.int32)
        b_max = bb - 1

        def do_chunk(k, _):
            c = wid + k * nw
            base = jnp.minimum(c * _C, n - _C)
            pltpu.sync_copy(x_hbm.at[pl.ds(base, _C), :], x_v)

            @plsc.parallel_loop(0, _G, 1, unroll=4)
            def grp_a(g):
                p = g * _L + iota
                x0 = plsc.load_gather(x_v, [p, zero16])
                x1 = plsc.load_gather(x_v, [p, one16])
                pos0 = jnp.zeros((_L,), jnp.int32)
                pos1 = jnp.zeros((_L,), jnp.int32)
                for s in (256, 128, 64, 32, 16, 8, 4, 2, 1):
                    k0 = plsc.load_gather(knots_v, [pos0 + (s - 1), zero16])
                    pos0 = jnp.where(k0 < x0, pos0 + s, pos0)
                    k1 = plsc.load_gather(knots_v, [pos1 + (s - 1), one16])
                    pos1 = jnp.where(k1 < x1, pos1 + s, pos1)
                i0 = jnp.clip(pos0 - 1, 0, b_max)
                i1 = jnp.clip(pos1 - 1, 0, b_max)
                t0 = x0 - plsc.load_gather(knots_v, [i0, zero16])
                t1 = x1 - plsc.load_gather(knots_v, [i1, one16])
                t0_v[pl.ds(g * _L, _L)] = t0
                t1_v[pl.ds(g * _L, _L)] = t1
                idx_v[pl.ds(g * _L, _L)] = i0 * bb + i1

            descs = []
            for s in range(_C // _STREAM):
                descs.append(pltpu.async_copy(
                    packed_hbm.at[idx_v.at[pl.ds(s * _STREAM, _STREAM)]],
                    rows_v.at[pl.ds(s * _STREAM, _STREAM)],
                    sem))
            for d in descs:
                d.wait()

            @plsc.parallel_loop(0, _G, 1, unroll=4)
            def grp_c(g):
                p = g * _L + iota
                t0 = t0_v[pl.ds(g * _L, _L)]
                t1 = t1_v[pl.ds(g * _L, _L)]
                acc = jnp.zeros((_L,), jnp.float32)
                for m in range(4):
                    a = []
                    for i in range(3):
                        col = m * 9 + i * 3
                        c0 = plsc.load_gather(rows_v, [p, jnp.full((_L,), col, jnp.int32)])
                        c1 = plsc.load_gather(rows_v, [p, jnp.full((_L,), col + 1, jnp.int32)])
                        c2 = plsc.load_gather(rows_v, [p, jnp.full((_L,), col + 2, jnp.int32)])
                        a.append((c2 * t1 + c1) * t1 + c0)
                    val = (a[2] * t0 + a[1]) * t0 + a[0]
                    wi = plsc.load_gather(rows_v, [p, jnp.full((_L,), 36 + m, jnp.int32)])
                    acc = acc + val * val * wi
                outc_v[pl.ds(g * _L, _L)] = _density_log(acc)

            pltpu.sync_copy(outc_v, out_hbm.at[pl.ds(base, _C)])
            return 0

        my_chunks = (num_chunks - wid + nw - 1) // nw
        lax.fori_loop(0, my_chunks, do_chunk, 0)

    return sc_kernel


def kernel(x, knots, poly_params, mixture_weights, integrals_2dgrid):
    n = x.shape[0]
    kn = knots.shape[0]
    bb = kn - 1
    coeff = jnp.transpose(poly_params[0].reshape(4, bb * bb, 9), (1, 0, 2))
    coeff = coeff.reshape(bb * bb, 36)
    wi = mixture_weights[0][:, None] / integrals_2dgrid[0].reshape(4, bb * bb)
    wi = jnp.transpose(wi, (1, 0))
    packed = jnp.concatenate(
        [coeff, wi, jnp.zeros((bb * bb, _ROW - 40), jnp.float32)], axis=1)
    return _make_sc_kernel(n, kn, bb)(x, knots, packed)
